# Initial kernel scaffold; baseline (speedup 1.0000x reference)
import jax
import jax.numpy as jnp
from jax.experimental import pallas as pl


def kernel(x, Wq, Wk, Wv, Wo):
    raise NotImplementedError("write your pallas kernel here")


# speedup vs baseline: 5.3616x; 5.3616x over previous
import functools

import jax
import jax.numpy as jnp
from jax.experimental import pallas as pl
from jax.experimental.pallas import tpu as pltpu

C = 384
H = 224
W = 224
S = 16
NH = H // S
NW = W // S
K = NH * NW
KP = 224
PB = S * W
NEG = -1e30
INV_SQRT_C = float(1.0 / (C ** 0.5))


def _pool_kernel(x_ref, poolw_ref, out_ref):
    s = jnp.sum(x_ref[...], axis=1)
    out_ref[0] = jax.lax.dot_general(
        s, poolw_ref[...], (((1,), (0,)), ((), ())),
        preferred_element_type=jnp.float32)


def _ssn_kernel(pix_ref, cent_ref, wk_ref, wv_ref, ks_ref, vs_ref,
                num_acc, den_acc):
    bh = pl.program_id(0)
    pixb = pix_ref[...]
    cent = cent_ref[...]
    dots = jax.lax.dot_general(
        cent, pixb, (((1,), (0,)), ((), ())),
        preferred_element_type=jnp.float32)
    cent_sq = jnp.sum(cent * cent, axis=1, keepdims=True)
    logits = 2.0 * dots - cent_sq
    ks2 = jax.lax.broadcasted_iota(jnp.int32, (KP, PB), 0)
    kh = ks2 // NW
    kw = ks2 % NW
    lp = jax.lax.broadcasted_iota(jnp.int32, (KP, PB), 1)
    bw = (lp % W) // S
    valid = ((jnp.abs(kh - bh) <= 1) & (jnp.abs(kw - bw) <= 1) & (ks2 < K))
    lm = jnp.where(valid, logits, NEG)
    m = jnp.max(lm, axis=0, keepdims=True)
    e = jnp.exp(lm - m)
    den = jnp.sum(e, axis=0, keepdims=True)
    aff = e / den
    contrib = jax.lax.dot_general(
        aff, pixb, (((1,), (1,)), ((), ())),
        preferred_element_type=jnp.float32)
    dcontrib = jnp.broadcast_to(
        jnp.sum(aff, axis=1, keepdims=True), (KP, 128))

    @pl.when(bh == 0)
    def _():
        num_acc[...] = contrib
        den_acc[...] = dcontrib

    @pl.when(bh > 0)
    def _():
        num_acc[...] += contrib
        den_acc[...] += dcontrib

    @pl.when(bh == NH - 1)
    def _():
        cent1 = num_acc[...] / (den_acc[...][:, :1] + 1e-16)
        ks_ref[...] = jnp.dot(cent1, wk_ref[...],
                              preferred_element_type=jnp.float32)
        vs_ref[...] = jnp.dot(cent1, wv_ref[...],
                              preferred_element_type=jnp.float32)


def _attn_kernel(pix_ref, wq_ref, wo_ref, ks_ref, vs_ref, y_ref):
    pixb = pix_ref[...]
    qT = jax.lax.dot_general(
        wq_ref[...], pixb, (((0,), (0,)), ((), ())),
        preferred_element_type=jnp.float32)
    logits = jax.lax.dot_general(
        ks_ref[...], qT, (((1,), (0,)), ((), ())),
        preferred_element_type=jnp.float32) * INV_SQRT_C
    ks2 = jax.lax.broadcasted_iota(jnp.int32, (KP, PB), 0)
    lm = jnp.where(ks2 < K, logits, NEG)
    m = jnp.max(lm, axis=0, keepdims=True)
    e = jnp.exp(lm - m)
    attnT = e / jnp.sum(e, axis=0, keepdims=True)
    outT = jax.lax.dot_general(
        vs_ref[...], attnT, (((0,), (0,)), ((), ())),
        preferred_element_type=jnp.float32)
    projT = jax.lax.dot_general(
        wo_ref[...], outT, (((0,), (0,)), ((), ())),
        preferred_element_type=jnp.float32)
    y_ref[...] = pixb + projT


@functools.partial(jax.jit, static_argnames=("interpret",))
def kernel(x, Wq, Wk, Wv, Wo, interpret=False):
    x3 = x.reshape(C, H, W)
    pix = x.reshape(C, H * W)
    poolw_np = (jnp.arange(W)[:, None] // S ==
                jnp.arange(NW)[None, :]).astype(jnp.float32) / (S * S)

    cent_rows = pl.pallas_call(
        _pool_kernel,
        grid=(NH,),
        in_specs=[
            pl.BlockSpec((C, S, W), lambda i: (0, i, 0)),
            pl.BlockSpec((W, NW), lambda i: (0, 0)),
        ],
        out_specs=pl.BlockSpec((1, C, NW), lambda i: (i, 0, 0)),
        out_shape=jax.ShapeDtypeStruct((NH, C, NW), jnp.float32),
        interpret=interpret,
    )(x3, poolw_np)
    cent0 = jnp.pad(cent_rows.transpose(0, 2, 1).reshape(K, C),
                    ((0, KP - K), (0, 0)))

    ks, vs = pl.pallas_call(
        _ssn_kernel,
        grid=(NH,),
        in_specs=[
            pl.BlockSpec((C, PB), lambda i: (0, i)),
            pl.BlockSpec((KP, C), lambda i: (0, 0)),
            pl.BlockSpec((C, C), lambda i: (0, 0)),
            pl.BlockSpec((C, C), lambda i: (0, 0)),
        ],
        out_specs=[
            pl.BlockSpec((KP, C), lambda i: (0, 0)),
            pl.BlockSpec((KP, C), lambda i: (0, 0)),
        ],
        out_shape=[
            jax.ShapeDtypeStruct((KP, C), jnp.float32),
            jax.ShapeDtypeStruct((KP, C), jnp.float32),
        ],
        scratch_shapes=[
            pltpu.VMEM((KP, C), jnp.float32),
            pltpu.VMEM((KP, 128), jnp.float32),
        ],
        interpret=interpret,
    )(pix, cent0, Wk, Wv)

    y = pl.pallas_call(
        _attn_kernel,
        grid=(NH,),
        in_specs=[
            pl.BlockSpec((C, PB), lambda i: (0, i)),
            pl.BlockSpec((C, C), lambda i: (0, 0)),
            pl.BlockSpec((C, C), lambda i: (0, 0)),
            pl.BlockSpec((KP, C), lambda i: (0, 0)),
            pl.BlockSpec((KP, C), lambda i: (0, 0)),
        ],
        out_specs=pl.BlockSpec((C, PB), lambda i: (0, i)),
        out_shape=jax.ShapeDtypeStruct((C, H * W), jnp.float32),
        compiler_params=pltpu.CompilerParams(
            dimension_semantics=("arbitrary",)),
        interpret=interpret,
    )(pix, Wq, Wo, ks, vs)

    return y.reshape(1, C, H, W)
